# baseline (speedup 1.0000x reference)
import functools

import jax
import jax.numpy as jnp
from jax import lax
from jax.experimental import pallas as pl
from jax.experimental.pallas import tpu as pltpu
from jax.experimental.pallas import tpu_sc as plsc

_B, _C, _K1, _D = 4096, 10, 6, 64
_NW = 32
_BPW = _B // _NW
_HALF = _BPW // 2
_LG = 16
_ND = _D // _LG
_ROWS = _B * _K1 * _LG // 128


_CH = 32
_NROUND = _BPW // _CH
_DP = 128


def _streams(total):
    out, off = [], 0
    while off < total:
        n = min(128, total - off)
        out.append((off, n))
        off += n
    return out


def _sc_scores(u_half, u_offs, w_half, w_offs, table2):
    mesh = plsc.VectorSubcoreMesh(core_axis_name="c", subcore_axis_name="s")

    @functools.partial(
        pl.kernel,
        out_type=jax.ShapeDtypeStruct((_B * _K1, _LG), jnp.float32),
        mesh=mesh,
        scratch_types=[
            pltpu.VMEM((_BPW * _C,), jnp.int32),
            pltpu.VMEM((_BPW * _K1,), jnp.int32),
            pltpu.VMEM((_BPW * _C * _LG,), jnp.int32),
            pltpu.VMEM((_BPW * _K1 * _LG,), jnp.int32),
            pltpu.VMEM((_CH * _C, _DP), jnp.float32),
            pltpu.VMEM((_CH * _K1, _DP), jnp.float32),
            pltpu.VMEM((_CH * _K1, _LG), jnp.float32),
            pltpu.SemaphoreType.DMA,
        ],
        compiler_params=pltpu.CompilerParams(needs_layout_passes=False),
    )
    def body(uh_hbm, uo_hbm, wh_hbm, wo_hbm, tab_hbm, out_hbm,
             uh_v, wh_v, uo_v, wo_v, u_rows, w_rows, out_v, sem):
        wid = lax.axis_index("s") * 2 + lax.axis_index("c")
        base = wid * _BPW
        pltpu.sync_copy(uh_hbm.at[pl.ds(base * _C, _BPW * _C)], uh_v)
        pltpu.sync_copy(wh_hbm.at[pl.ds(base * _K1, _BPW * _K1)], wh_v)
        pltpu.sync_copy(
            uo_hbm.at[pl.ds(base * _C * _LG, _BPW * _C * _LG)], uo_v)
        pltpu.sync_copy(
            wo_hbm.at[pl.ds(base * _K1 * _LG, _BPW * _K1 * _LG)], wo_v)

        for rnd in range(_NROUND):
            off = rnd * _CH
            copies = []
            for (so, sn) in _streams(_CH * _C):
                copies.append(pltpu.async_copy(
                    tab_hbm.at[uh_v.at[pl.ds(off * _C + so, sn)]],
                    u_rows.at[pl.ds(so, sn)], sem))
            for (so, sn) in _streams(_CH * _K1):
                copies.append(pltpu.async_copy(
                    tab_hbm.at[wh_v.at[pl.ds(off * _K1 + so, sn)]],
                    w_rows.at[pl.ds(so, sn)], sem))
            for cp in copies:
                cp.wait()

            zeros16 = jnp.zeros((_LG,), jnp.int32)

            def elem(e, carry):
                ucols = [uo_v[pl.ds(((off + e) * _C + c) * _LG, _LG)]
                         for c in range(_C)]
                wcols = [wo_v[pl.ds(((off + e) * _K1 + t) * _LG, _LG)]
                         for t in range(_K1)]
                urow = [zeros16 + (e * _C + c) for c in range(_C)]
                wrow = [zeros16 + (e * _K1 + t) for t in range(_K1)]
                accs = []
                for d in range(_ND):
                    a = plsc.load_gather(
                        u_rows, [urow[0], ucols[0] + (d * _LG)])
                    for c in range(1, _C):
                        a = a + plsc.load_gather(
                            u_rows, [urow[c], ucols[c] + (d * _LG)])
                    accs.append(a)
                for t in range(_K1):
                    p = accs[0] * plsc.load_gather(
                        w_rows, [wrow[t], wcols[t]])
                    for d in range(1, _ND):
                        p = p + accs[d] * plsc.load_gather(
                            w_rows, [wrow[t], wcols[t] + (d * _LG)])
                    if t > 0:
                        p = -p
                    out_v[e * _K1 + t, pl.ds(0, _LG)] = p
                return carry

            lax.fori_loop(0, _CH, elem, 0)
            pltpu.sync_copy(
                out_v, out_hbm.at[pl.ds((base + off) * _K1, _CH * _K1)])

    return body(u_half, u_offs, w_half, w_offs, table2)


def _tc_loss_body(x_ref, o_ref):
    x = x_ref[...]
    col = lax.broadcasted_iota(jnp.int32, (128, 8), 0) // _LG
    grp = lax.broadcasted_iota(jnp.int32, (128, 8), 1)
    sel = (col == grp).astype(jnp.float32)
    s = jnp.dot(x, sel, preferred_element_type=jnp.float32)
    ls = jnp.minimum(s, 0.0) - jnp.log1p(jnp.exp(-jnp.abs(s)))
    o_ref[0, 0] = -jnp.sum(ls)


def _tc_loss(lanes_flat):
    return pl.pallas_call(
        _tc_loss_body,
        out_shape=jax.ShapeDtypeStruct((1, 1), jnp.float32),
        in_specs=[pl.BlockSpec(memory_space=pltpu.VMEM)],
        out_specs=pl.BlockSpec(memory_space=pltpu.SMEM),
    )(lanes_flat)


def kernel(pos_u, pos_w, neg_w, u_weight, w_weight):
    u_idx = pos_u.astype(jnp.int32).reshape(_B * _C)
    w_idx = jnp.concatenate(
        [pos_w.astype(jnp.int32)[:, None], neg_w.astype(jnp.int32)],
        axis=1).reshape(_B * _K1)
    v = u_weight.shape[0]
    table2 = jnp.concatenate([u_weight, w_weight], axis=0).reshape(v, _DP)
    u_half = u_idx >> 1
    w_half = (v >> 1) + (w_idx >> 1)
    lanes16 = jnp.arange(_LG, dtype=jnp.int32)[None, :]
    u_offs = (((u_idx & 1) * _D)[:, None] + lanes16).reshape(_B * _C * _LG)
    w_offs = (((w_idx & 1) * _D)[:, None] + lanes16).reshape(_B * _K1 * _LG)
    lanes = _sc_scores(u_half, u_offs, w_half, w_offs, table2)
    return _tc_loss(lanes.reshape(_ROWS, 128))[0, 0]


# speedup vs baseline: 1.0624x; 1.0624x over previous
import functools

import jax
import jax.numpy as jnp
from jax import lax
from jax.experimental import pallas as pl
from jax.experimental.pallas import tpu as pltpu
from jax.experimental.pallas import tpu_sc as plsc

_B, _C, _K1, _D = 4096, 10, 6, 64
_NW = 32
_BPW = _B // _NW
_HALF = _BPW // 2
_LG = 16
_ND = _D // _LG
_ROWS = _B * _K1 * _LG // 128


_CH = 32
_NROUND = _BPW // _CH
_DP = 128


def _streams(total):
    out, off = [], 0
    while off < total:
        n = min(128, total - off)
        out.append((off, n))
        off += n
    return out


def _sc_scores(u_idx, w_idx, u128, w128):
    mesh = plsc.VectorSubcoreMesh(core_axis_name="c", subcore_axis_name="s")

    @functools.partial(
        pl.kernel,
        out_type=jax.ShapeDtypeStruct((_B * _K1, _LG), jnp.float32),
        mesh=mesh,
        scratch_types=[
            pltpu.VMEM((_BPW * _C,), jnp.int32),
            pltpu.VMEM((_BPW * _K1,), jnp.int32),
            pltpu.VMEM((_CH * _C, _DP), jnp.float32),
            pltpu.VMEM((_CH * _K1, _DP), jnp.float32),
            pltpu.VMEM((_CH * _K1, _LG), jnp.float32),
            pltpu.SemaphoreType.DMA,
        ],
    )
    def body(u_idx_hbm, w_idx_hbm, uw_hbm, ww_hbm, out_hbm,
             u_idx_v, w_idx_v, u_rows, w_rows, out_v, sem):
        wid = lax.axis_index("s") * 2 + lax.axis_index("c")
        base = wid * _BPW
        pltpu.sync_copy(u_idx_hbm.at[pl.ds(base * _C, _BPW * _C)], u_idx_v)
        pltpu.sync_copy(w_idx_hbm.at[pl.ds(base * _K1, _BPW * _K1)], w_idx_v)

        for rnd in range(_NROUND):
            off = rnd * _CH
            copies = []
            for (so, sn) in _streams(_CH * _C):
                copies.append(pltpu.async_copy(
                    uw_hbm.at[u_idx_v.at[pl.ds(off * _C + so, sn)]],
                    u_rows.at[pl.ds(so, sn)], sem))
            for (so, sn) in _streams(_CH * _K1):
                copies.append(pltpu.async_copy(
                    ww_hbm.at[w_idx_v.at[pl.ds(off * _K1 + so, sn)]],
                    w_rows.at[pl.ds(so, sn)], sem))
            for cp in copies:
                cp.wait()

            def elem(e, carry):
                accs = []
                for d in range(_ND):
                    a = u_rows[e * _C, pl.ds(d * _LG, _LG)]
                    for c in range(1, _C):
                        a = a + u_rows[e * _C + c, pl.ds(d * _LG, _LG)]
                    accs.append(a)
                for t in range(_K1):
                    p = accs[0] * w_rows[e * _K1 + t, pl.ds(0, _LG)]
                    for d in range(1, _ND):
                        p = p + accs[d] * w_rows[e * _K1 + t,
                                                 pl.ds(d * _LG, _LG)]
                    if t > 0:
                        p = -p
                    out_v[e * _K1 + t, pl.ds(0, _LG)] = p
                return carry

            lax.fori_loop(0, _CH, elem, 0)
            pltpu.sync_copy(
                out_v, out_hbm.at[pl.ds((base + off) * _K1, _CH * _K1)])

    return body(u_idx, w_idx, u128, w128)


_TW = 1024


def _tc_relayout_body(xt_ref, o_ref):
    y = jnp.transpose(xt_ref[...], (1, 0))
    o_ref[...] = jnp.pad(y, ((0, 0), (0, _DP - _D)))


def _tc_relayout(table):
    v = table.shape[0]
    grid = (v + _TW - 1) // _TW
    return pl.pallas_call(
        _tc_relayout_body,
        grid=(grid,),
        in_specs=[pl.BlockSpec((_D, _TW), lambda j: (0, j))],
        out_specs=pl.BlockSpec((_TW, _DP), lambda j: (j, 0)),
        out_shape=jax.ShapeDtypeStruct((v, _DP), jnp.float32),
    )(table.T)


def _tc_loss_body(x_ref, o_ref):
    x = x_ref[...]
    col = lax.broadcasted_iota(jnp.int32, (128, 8), 0) // _LG
    grp = lax.broadcasted_iota(jnp.int32, (128, 8), 1)
    sel = (col == grp).astype(jnp.float32)
    s = jnp.dot(x, sel, preferred_element_type=jnp.float32)
    ls = jnp.minimum(s, 0.0) - jnp.log1p(jnp.exp(-jnp.abs(s)))
    o_ref[0, 0] = -jnp.sum(ls)


def _tc_loss(lanes_flat):
    return pl.pallas_call(
        _tc_loss_body,
        out_shape=jax.ShapeDtypeStruct((1, 1), jnp.float32),
        in_specs=[pl.BlockSpec(memory_space=pltpu.VMEM)],
        out_specs=pl.BlockSpec(memory_space=pltpu.SMEM),
    )(lanes_flat)


def kernel(pos_u, pos_w, neg_w, u_weight, w_weight):
    u_idx = pos_u.astype(jnp.int32).reshape(_B * _C)
    w_idx = jnp.concatenate(
        [pos_w.astype(jnp.int32)[:, None], neg_w.astype(jnp.int32)],
        axis=1).reshape(_B * _K1)
    u128 = _tc_relayout(u_weight)
    w128 = _tc_relayout(w_weight)
    lanes = _sc_scores(u_idx, w_idx, u128, w128)
    return _tc_loss(lanes.reshape(_ROWS, 128))[0, 0]


# speedup vs baseline: 2.0944x; 1.9713x over previous
import functools

import jax
import jax.numpy as jnp
from jax import lax
from jax.experimental import pallas as pl
from jax.experimental.pallas import tpu as pltpu
from jax.experimental.pallas import tpu_sc as plsc

_B, _C, _K1, _D = 4096, 10, 6, 64
_NW = 32
_BPW = _B // _NW
_HALF = _BPW // 2
_LG = 16
_ND = _D // _LG
_ROWS = _B * _K1 * _LG // 128


_CH = 32
_NROUND = _BPW // _CH
_DP = 128


def _streams(total):
    out, off = [], 0
    while off < total:
        n = min(128, total - off)
        out.append((off, n))
        off += n
    return out


def _sc_scores(u_idx, w_idx, u128, w128):
    mesh = plsc.VectorSubcoreMesh(core_axis_name="c", subcore_axis_name="s")

    @functools.partial(
        pl.kernel,
        out_type=jax.ShapeDtypeStruct((_B * _K1, _LG), jnp.float32),
        mesh=mesh,
        scratch_types=[
            pltpu.VMEM((_BPW * _C,), jnp.int32),
            pltpu.VMEM((_BPW * _K1,), jnp.int32),
            pltpu.VMEM((_CH * _C, _DP), jnp.float32),
            pltpu.VMEM((_CH * _K1, _DP), jnp.float32),
            pltpu.VMEM((_CH * _K1, _LG), jnp.float32),
            pltpu.SemaphoreType.DMA,
        ],
    )
    def body(u_idx_hbm, w_idx_hbm, uw_hbm, ww_hbm, out_hbm,
             u_idx_v, w_idx_v, u_rows, w_rows, out_v, sem):
        wid = lax.axis_index("s") * 2 + lax.axis_index("c")
        base = wid * _BPW
        pltpu.sync_copy(u_idx_hbm.at[pl.ds(base * _C, _BPW * _C)], u_idx_v)
        pltpu.sync_copy(w_idx_hbm.at[pl.ds(base * _K1, _BPW * _K1)], w_idx_v)

        for rnd in range(_NROUND):
            off = rnd * _CH
            copies = []
            for (so, sn) in _streams(_CH * _C):
                copies.append(pltpu.async_copy(
                    uw_hbm.at[u_idx_v.at[pl.ds(off * _C + so, sn)]],
                    u_rows.at[pl.ds(so, sn)], sem))
            for (so, sn) in _streams(_CH * _K1):
                copies.append(pltpu.async_copy(
                    ww_hbm.at[w_idx_v.at[pl.ds(off * _K1 + so, sn)]],
                    w_rows.at[pl.ds(so, sn)], sem))
            for cp in copies:
                cp.wait()

            def elem(e, carry):
                accs = []
                for d in range(_ND):
                    a = u_rows[e * _C, pl.ds(d * _LG, _LG)]
                    for c in range(1, _C):
                        a = a + u_rows[e * _C + c, pl.ds(d * _LG, _LG)]
                    accs.append(a)
                for t in range(_K1):
                    p = accs[0] * w_rows[e * _K1 + t, pl.ds(0, _LG)]
                    for d in range(1, _ND):
                        p = p + accs[d] * w_rows[e * _K1 + t,
                                                 pl.ds(d * _LG, _LG)]
                    if t > 0:
                        p = -p
                    out_v[e * _K1 + t, pl.ds(0, _LG)] = p
                return carry

            lax.fori_loop(0, _CH, elem, 0)
            pltpu.sync_copy(
                out_v, out_hbm.at[pl.ds((base + off) * _K1, _CH * _K1)])

    return body(u_idx, w_idx, u128, w128)


_TW = 4096


def _tc_relayout_body(xt_ref, o_ref):
    d_ids = lax.broadcasted_iota(jnp.int32, (_D, _DP), 0)
    p_ids = lax.broadcasted_iota(jnp.int32, (_D, _DP), 1)
    sel = (d_ids == p_ids).astype(jnp.float32)
    o_ref[...] = jax.lax.dot_general(
        xt_ref[...], sel, (((0,), (0,)), ((), ())),
        preferred_element_type=jnp.float32)


def _tc_relayout(table):
    v = table.shape[0]
    grid = (v + _TW - 1) // _TW
    return pl.pallas_call(
        _tc_relayout_body,
        grid=(grid,),
        in_specs=[pl.BlockSpec((_D, _TW), lambda j: (0, j))],
        out_specs=pl.BlockSpec((_TW, _DP), lambda j: (j, 0)),
        out_shape=jax.ShapeDtypeStruct((v, _DP), jnp.float32),
    )(table.T)


def _tc_loss_body(x_ref, o_ref):
    x = x_ref[...]
    col = lax.broadcasted_iota(jnp.int32, (128, 8), 0) // _LG
    grp = lax.broadcasted_iota(jnp.int32, (128, 8), 1)
    sel = (col == grp).astype(jnp.float32)
    s = jnp.dot(x, sel, preferred_element_type=jnp.float32)
    ls = jnp.minimum(s, 0.0) - jnp.log1p(jnp.exp(-jnp.abs(s)))
    o_ref[0, 0] = -jnp.sum(ls)


def _tc_loss(lanes_flat):
    return pl.pallas_call(
        _tc_loss_body,
        out_shape=jax.ShapeDtypeStruct((1, 1), jnp.float32),
        in_specs=[pl.BlockSpec(memory_space=pltpu.VMEM)],
        out_specs=pl.BlockSpec(memory_space=pltpu.SMEM),
    )(lanes_flat)


def kernel(pos_u, pos_w, neg_w, u_weight, w_weight):
    u_idx = pos_u.astype(jnp.int32).reshape(_B * _C)
    w_idx = jnp.concatenate(
        [pos_w.astype(jnp.int32)[:, None], neg_w.astype(jnp.int32)],
        axis=1).reshape(_B * _K1)
    u128 = _tc_relayout(u_weight)
    w128 = _tc_relayout(w_weight)
    lanes = _sc_scores(u_idx, w_idx, u128, w128)
    return _tc_loss(lanes.reshape(_ROWS, 128))[0, 0]


# speedup vs baseline: 2.6499x; 1.2653x over previous
import functools

import jax
import jax.numpy as jnp
from jax import lax
from jax.experimental import pallas as pl
from jax.experimental.pallas import tpu as pltpu
from jax.experimental.pallas import tpu_sc as plsc

_B, _C, _K1, _D = 4096, 10, 6, 64
_NW = 32
_BPW = _B // _NW
_HALF = _BPW // 2
_LG = 16
_ND = _D // _LG
_ROWS = _B * _K1 * _LG // 128


_CH = 32
_NROUND = _BPW // _CH
_DP = 128


def _streams(total):
    out, off = [], 0
    while off < total:
        n = min(128, total - off)
        out.append((off, n))
        off += n
    return out


def _sc_scores(u_half, u_offs, w_half, w_offs, u2, w2):
    mesh = plsc.VectorSubcoreMesh(core_axis_name="c", subcore_axis_name="s")

    @functools.partial(
        pl.kernel,
        out_type=jax.ShapeDtypeStruct((_B * _K1, _LG), jnp.float32),
        mesh=mesh,
        scratch_types=[
            pltpu.VMEM((_BPW * _C,), jnp.int32),
            pltpu.VMEM((_BPW * _K1,), jnp.int32),
            pltpu.VMEM((_BPW * _C * _LG,), jnp.int32),
            pltpu.VMEM((_BPW * _K1 * _LG,), jnp.int32),
            pltpu.VMEM((_CH * _C, _DP), jnp.float32),
            pltpu.VMEM((_CH * _K1, _DP), jnp.float32),
            pltpu.VMEM((_CH * _K1, _LG), jnp.float32),
            pltpu.SemaphoreType.DMA,
        ],
        compiler_params=pltpu.CompilerParams(needs_layout_passes=False),
    )
    def body(uh_hbm, uo_hbm, wh_hbm, wo_hbm, ut_hbm, wt_hbm, out_hbm,
             uh_v, wh_v, uo_v, wo_v, u_rows, w_rows, out_v, sem):
        wid = lax.axis_index("s") * 2 + lax.axis_index("c")
        base = wid * _BPW
        pltpu.sync_copy(uh_hbm.at[pl.ds(base * _C, _BPW * _C)], uh_v)
        pltpu.sync_copy(wh_hbm.at[pl.ds(base * _K1, _BPW * _K1)], wh_v)
        pltpu.sync_copy(
            uo_hbm.at[pl.ds(base * _C * _LG, _BPW * _C * _LG)], uo_v)
        pltpu.sync_copy(
            wo_hbm.at[pl.ds(base * _K1 * _LG, _BPW * _K1 * _LG)], wo_v)

        for rnd in range(_NROUND):
            off = rnd * _CH
            copies = []
            for (so, sn) in _streams(_CH * _C):
                copies.append(pltpu.async_copy(
                    ut_hbm.at[uh_v.at[pl.ds(off * _C + so, sn)]],
                    u_rows.at[pl.ds(so, sn)], sem))
            for (so, sn) in _streams(_CH * _K1):
                copies.append(pltpu.async_copy(
                    wt_hbm.at[wh_v.at[pl.ds(off * _K1 + so, sn)]],
                    w_rows.at[pl.ds(so, sn)], sem))
            for cp in copies:
                cp.wait()

            zeros16 = jnp.zeros((_LG,), jnp.int32)

            def elem(e, carry):
                ucols = [uo_v[pl.ds(((off + e) * _C + c) * _LG, _LG)]
                         for c in range(_C)]
                wcols = [wo_v[pl.ds(((off + e) * _K1 + t) * _LG, _LG)]
                         for t in range(_K1)]
                urow = [zeros16 + (e * _C + c) for c in range(_C)]
                wrow = [zeros16 + (e * _K1 + t) for t in range(_K1)]
                accs = []
                for d in range(_ND):
                    a = plsc.load_gather(
                        u_rows, [urow[0], ucols[0] + (d * _LG)])
                    for c in range(1, _C):
                        a = a + plsc.load_gather(
                            u_rows, [urow[c], ucols[c] + (d * _LG)])
                    accs.append(a)
                for t in range(_K1):
                    p = accs[0] * plsc.load_gather(
                        w_rows, [wrow[t], wcols[t]])
                    for d in range(1, _ND):
                        p = p + accs[d] * plsc.load_gather(
                            w_rows, [wrow[t], wcols[t] + (d * _LG)])
                    if t > 0:
                        p = -p
                    out_v[e * _K1 + t, pl.ds(0, _LG)] = p
                return carry

            lax.fori_loop(0, _CH, elem, 0)
            pltpu.sync_copy(
                out_v, out_hbm.at[pl.ds((base + off) * _K1, _CH * _K1)])

    return body(u_half, u_offs, w_half, w_offs, u2, w2)


_TW = 4096
_SPLIT = 512000


def _tc_relayout_body(xa_ref, xb_ref, o_ref):
    d_ids = lax.broadcasted_iota(jnp.int32, (_D, _DP), 0)
    p_ids = lax.broadcasted_iota(jnp.int32, (_D, _DP), 1)
    e_lo = (d_ids == p_ids).astype(jnp.float32)
    e_hi = (d_ids == (p_ids - _D)).astype(jnp.float32)
    o_ref[...] = (
        jax.lax.dot_general(
            xa_ref[...], e_lo, (((0,), (0,)), ((), ())),
            preferred_element_type=jnp.float32)
        + jax.lax.dot_general(
            xb_ref[...], e_hi, (((0,), (0,)), ((), ())),
            preferred_element_type=jnp.float32))


def _tc_relayout(table):
    nblk = _SPLIT // _TW
    last = (table.shape[0] - 1) // _TW
    return pl.pallas_call(
        _tc_relayout_body,
        grid=(nblk,),
        in_specs=[
            pl.BlockSpec((_D, _TW), lambda j: (0, j)),
            pl.BlockSpec((_D, _TW),
                         lambda j: (0, jnp.minimum(j + nblk, last))),
        ],
        out_specs=pl.BlockSpec((_TW, _DP), lambda j: (j, 0)),
        out_shape=jax.ShapeDtypeStruct((_SPLIT, _DP), jnp.float32),
    )(table.T, table.T)


def _tc_loss_body(x_ref, o_ref):
    x = x_ref[...]
    col = lax.broadcasted_iota(jnp.int32, (128, 8), 0) // _LG
    grp = lax.broadcasted_iota(jnp.int32, (128, 8), 1)
    sel = (col == grp).astype(jnp.float32)
    s = jnp.dot(x, sel, preferred_element_type=jnp.float32)
    ls = jnp.minimum(s, 0.0) - jnp.log1p(jnp.exp(-jnp.abs(s)))
    o_ref[0, 0] = -jnp.sum(ls)


def _tc_loss(lanes_flat):
    return pl.pallas_call(
        _tc_loss_body,
        out_shape=jax.ShapeDtypeStruct((1, 1), jnp.float32),
        in_specs=[pl.BlockSpec(memory_space=pltpu.VMEM)],
        out_specs=pl.BlockSpec(memory_space=pltpu.SMEM),
    )(lanes_flat)


def kernel(pos_u, pos_w, neg_w, u_weight, w_weight):
    u_idx = pos_u.astype(jnp.int32).reshape(_B * _C)
    w_idx = jnp.concatenate(
        [pos_w.astype(jnp.int32)[:, None], neg_w.astype(jnp.int32)],
        axis=1).reshape(_B * _K1)
    u2 = _tc_relayout(u_weight)
    w2 = _tc_relayout(w_weight)
    u_half = jnp.where(u_idx >= _SPLIT, u_idx - _SPLIT, u_idx)
    w_half = jnp.where(w_idx >= _SPLIT, w_idx - _SPLIT, w_idx)
    lanes16 = jnp.arange(_LG, dtype=jnp.int32)[None, :]
    u_offs = (jnp.where(u_idx >= _SPLIT, _D, 0)[:, None]
              + lanes16).reshape(_B * _C * _LG)
    w_offs = (jnp.where(w_idx >= _SPLIT, _D, 0)[:, None]
              + lanes16).reshape(_B * _K1 * _LG)
    lanes = _sc_scores(u_half, u_offs, w_half, w_offs, u2, w2)
    return _tc_loss(lanes.reshape(_ROWS, 128))[0, 0]


# speedup vs baseline: 3.1545x; 1.1904x over previous
import functools

import jax
import jax.numpy as jnp
from jax import lax
from jax.experimental import pallas as pl
from jax.experimental.pallas import tpu as pltpu
from jax.experimental.pallas import tpu_sc as plsc

_B, _C, _K1, _D = 4096, 10, 6, 64
_NW = 32
_BPW = _B // _NW
_HALF = _BPW // 2
_LG = 16
_ND = _D // _LG
_ROWS = _B * _K1 * _LG // 128


_CH = 32
_NROUND = _BPW // _CH
_DP = 128


def _streams(total):
    out, off = [], 0
    while off < total:
        n = min(128, total - off)
        out.append((off, n))
        off += n
    return out


def _sc_scores(u_half, u_offs, w_half, w_offs, u2, w2):
    mesh = plsc.VectorSubcoreMesh(core_axis_name="c", subcore_axis_name="s")

    @functools.partial(
        pl.kernel,
        out_type=jax.ShapeDtypeStruct((_B * _K1, _LG), jnp.float32),
        mesh=mesh,
        scratch_types=[
            pltpu.VMEM((_BPW * _C,), jnp.int32),
            pltpu.VMEM((_BPW * _K1,), jnp.int32),
            pltpu.VMEM((_BPW * _C * _LG,), jnp.int32),
            pltpu.VMEM((_BPW * _K1 * _LG,), jnp.int32),
            pltpu.VMEM((_CH * _C, _DP), jnp.float32),
            pltpu.VMEM((_CH * _K1, _DP), jnp.float32),
            pltpu.VMEM((_CH * _K1, _LG), jnp.float32),
            pltpu.SemaphoreType.DMA,
        ],
        compiler_params=pltpu.CompilerParams(needs_layout_passes=False),
    )
    def body(uh_hbm, uo_hbm, wh_hbm, wo_hbm, ut_hbm, wt_hbm, out_hbm,
             uh_v, wh_v, uo_v, wo_v, u_rows, w_rows, out_v, sem):
        wid = lax.axis_index("s") * 2 + lax.axis_index("c")
        base = wid * _BPW
        pltpu.sync_copy(uh_hbm.at[pl.ds(base * _C, _BPW * _C)], uh_v)
        pltpu.sync_copy(wh_hbm.at[pl.ds(base * _K1, _BPW * _K1)], wh_v)
        pltpu.sync_copy(
            uo_hbm.at[pl.ds(base * _C * _LG, _BPW * _C * _LG)], uo_v)
        pltpu.sync_copy(
            wo_hbm.at[pl.ds(base * _K1 * _LG, _BPW * _K1 * _LG)], wo_v)

        for rnd in range(_NROUND):
            off = rnd * _CH
            copies = []
            for (so, sn) in _streams(_CH * _C):
                copies.append(pltpu.async_copy(
                    ut_hbm.at[uh_v.at[pl.ds(off * _C + so, sn)]],
                    u_rows.at[pl.ds(so, sn)], sem))
            for (so, sn) in _streams(_CH * _K1):
                copies.append(pltpu.async_copy(
                    wt_hbm.at[wh_v.at[pl.ds(off * _K1 + so, sn)]],
                    w_rows.at[pl.ds(so, sn)], sem))
            for cp in copies:
                cp.wait()

            zeros16 = jnp.zeros((_LG,), jnp.int32)

            def elem(e, carry):
                ucols = [uo_v[pl.ds(((off + e) * _C + c) * _LG, _LG)]
                         for c in range(_C)]
                wcols = [wo_v[pl.ds(((off + e) * _K1 + t) * _LG, _LG)]
                         for t in range(_K1)]
                urow = [zeros16 + (e * _C + c) for c in range(_C)]
                wrow = [zeros16 + (e * _K1 + t) for t in range(_K1)]
                accs = []
                for d in range(_ND):
                    a = plsc.load_gather(
                        u_rows, [urow[0], ucols[0] + (d * _LG)])
                    for c in range(1, _C):
                        a = a + plsc.load_gather(
                            u_rows, [urow[c], ucols[c] + (d * _LG)])
                    accs.append(a)
                for t in range(_K1):
                    p = accs[0] * plsc.load_gather(
                        w_rows, [wrow[t], wcols[t]])
                    for d in range(1, _ND):
                        p = p + accs[d] * plsc.load_gather(
                            w_rows, [wrow[t], wcols[t] + (d * _LG)])
                    if t > 0:
                        p = -p
                    out_v[e * _K1 + t, pl.ds(0, _LG)] = p
                return carry

            lax.fori_loop(0, _CH, elem, 0)
            pltpu.sync_copy(
                out_v, out_hbm.at[pl.ds((base + off) * _K1, _CH * _K1)])

    return body(u_half, u_offs, w_half, w_offs, u2, w2)


_TW = 10240
_SPLIT = 512000


def _tc_relayout_body(xa_ref, xb_ref, o_ref):
    d_ids = lax.broadcasted_iota(jnp.int32, (_D, _DP), 0)
    p_ids = lax.broadcasted_iota(jnp.int32, (_D, _DP), 1)
    e_lo = (d_ids == p_ids).astype(jnp.float32)
    e_hi = (d_ids == (p_ids - _D)).astype(jnp.float32)
    o_ref[...] = (
        jax.lax.dot_general(
            xa_ref[...], e_lo, (((0,), (0,)), ((), ())),
            preferred_element_type=jnp.float32)
        + jax.lax.dot_general(
            xb_ref[...], e_hi, (((0,), (0,)), ((), ())),
            preferred_element_type=jnp.float32))


def _tc_relayout(table):
    nblk = _SPLIT // _TW
    last = (table.shape[0] - 1) // _TW
    return pl.pallas_call(
        _tc_relayout_body,
        grid=(nblk,),
        in_specs=[
            pl.BlockSpec((_D, _TW), lambda j: (0, j)),
            pl.BlockSpec((_D, _TW),
                         lambda j: (0, jnp.minimum(j + nblk, last))),
        ],
        out_specs=pl.BlockSpec((_TW, _DP), lambda j: (j, 0)),
        out_shape=jax.ShapeDtypeStruct((_SPLIT, _DP), jnp.float32),
    )(table.T, table.T)


def _tc_loss_body(x_ref, o_ref):
    x = x_ref[...]
    col = lax.broadcasted_iota(jnp.int32, (128, 8), 0) // _LG
    grp = lax.broadcasted_iota(jnp.int32, (128, 8), 1)
    sel = (col == grp).astype(jnp.float32)
    s = jnp.dot(x, sel, preferred_element_type=jnp.float32)
    ls = jnp.minimum(s, 0.0) - jnp.log1p(jnp.exp(-jnp.abs(s)))
    o_ref[0, 0] = -jnp.sum(ls)


def _tc_loss(lanes_flat):
    return pl.pallas_call(
        _tc_loss_body,
        out_shape=jax.ShapeDtypeStruct((1, 1), jnp.float32),
        in_specs=[pl.BlockSpec(memory_space=pltpu.VMEM)],
        out_specs=pl.BlockSpec(memory_space=pltpu.SMEM),
    )(lanes_flat)


def kernel(pos_u, pos_w, neg_w, u_weight, w_weight):
    u_idx = pos_u.astype(jnp.int32).reshape(_B * _C)
    w_idx = jnp.concatenate(
        [pos_w.astype(jnp.int32)[:, None], neg_w.astype(jnp.int32)],
        axis=1).reshape(_B * _K1)
    u2 = _tc_relayout(u_weight)
    w2 = _tc_relayout(w_weight)
    u_half = jnp.where(u_idx >= _SPLIT, u_idx - _SPLIT, u_idx)
    w_half = jnp.where(w_idx >= _SPLIT, w_idx - _SPLIT, w_idx)
    lanes16 = jnp.arange(_LG, dtype=jnp.int32)[None, :]
    u_offs = (jnp.where(u_idx >= _SPLIT, _D, 0)[:, None]
              + lanes16).reshape(_B * _C * _LG)
    w_offs = (jnp.where(w_idx >= _SPLIT, _D, 0)[:, None]
              + lanes16).reshape(_B * _K1 * _LG)
    lanes = _sc_scores(u_half, u_offs, w_half, w_offs, u2, w2)
    return _tc_loss(lanes.reshape(_ROWS, 128))[0, 0]


# speedup vs baseline: 3.3481x; 1.0614x over previous
import functools

import jax
import jax.numpy as jnp
from jax import lax
from jax.experimental import pallas as pl
from jax.experimental.pallas import tpu as pltpu
from jax.experimental.pallas import tpu_sc as plsc

_B, _C, _K1, _D = 4096, 10, 6, 64
_NW = 32
_BPW = _B // _NW
_HALF = _BPW // 2
_LG = 16
_ND = _D // _LG
_ROWS = _B * _K1 * _LG // 128


_CH = 32
_NROUND = _BPW // _CH
_DP = 128


def _streams(total):
    out, off = [], 0
    while off < total:
        n = min(128, total - off)
        out.append((off, n))
        off += n
    return out


def _sc_scores(u_half, u_offs, w_half, w_offs, u2, w2):
    mesh = plsc.VectorSubcoreMesh(core_axis_name="c", subcore_axis_name="s")

    @functools.partial(
        pl.kernel,
        out_type=jax.ShapeDtypeStruct((_B * _K1, _LG), jnp.float32),
        mesh=mesh,
        scratch_types=[
            pltpu.VMEM((_BPW * _C,), jnp.int32),
            pltpu.VMEM((_BPW * _K1,), jnp.int32),
            pltpu.VMEM((_BPW * _C * _LG,), jnp.int32),
            pltpu.VMEM((_BPW * _K1 * _LG,), jnp.int32),
            pltpu.VMEM((_CH * _C, _DP), jnp.float32),
            pltpu.VMEM((_CH * _K1, _DP), jnp.float32),
            pltpu.VMEM((_CH * _K1, _LG), jnp.float32),
            pltpu.SemaphoreType.DMA,
        ],
        compiler_params=pltpu.CompilerParams(needs_layout_passes=False),
    )
    def body(uh_hbm, uo_hbm, wh_hbm, wo_hbm, ut_hbm, wt_hbm, out_hbm,
             uh_v, wh_v, uo_v, wo_v, u_rows, w_rows, out_v, sem):
        wid = lax.axis_index("s") * 2 + lax.axis_index("c")
        base = wid * _BPW
        pltpu.sync_copy(uh_hbm.at[pl.ds(base * _C, _BPW * _C)], uh_v)
        pltpu.sync_copy(wh_hbm.at[pl.ds(base * _K1, _BPW * _K1)], wh_v)
        pltpu.sync_copy(
            uo_hbm.at[pl.ds(base * _C * _LG, _BPW * _C * _LG)], uo_v)
        pltpu.sync_copy(
            wo_hbm.at[pl.ds(base * _K1 * _LG, _BPW * _K1 * _LG)], wo_v)

        for rnd in range(_NROUND):
            off = rnd * _CH
            copies = []
            for (so, sn) in _streams(_CH * _C):
                copies.append(pltpu.async_copy(
                    ut_hbm.at[uh_v.at[pl.ds(off * _C + so, sn)]],
                    u_rows.at[pl.ds(so, sn)], sem))
            for (so, sn) in _streams(_CH * _K1):
                copies.append(pltpu.async_copy(
                    wt_hbm.at[wh_v.at[pl.ds(off * _K1 + so, sn)]],
                    w_rows.at[pl.ds(so, sn)], sem))
            for cp in copies:
                cp.wait()

            zeros16 = jnp.zeros((_LG,), jnp.int32)

            def elem(e, carry):
                ucols = [uo_v[pl.ds(((off + e) * _C + c) * _LG, _LG)]
                         for c in range(_C)]
                wcols = [wo_v[pl.ds(((off + e) * _K1 + t) * _LG, _LG)]
                         for t in range(_K1)]
                urow = [zeros16 + (e * _C + c) for c in range(_C)]
                wrow = [zeros16 + (e * _K1 + t) for t in range(_K1)]
                accs = []
                for d in range(_ND):
                    a = plsc.load_gather(
                        u_rows, [urow[0], ucols[0] + (d * _LG)])
                    for c in range(1, _C):
                        a = a + plsc.load_gather(
                            u_rows, [urow[c], ucols[c] + (d * _LG)])
                    accs.append(a)
                for t in range(_K1):
                    p = accs[0] * plsc.load_gather(
                        w_rows, [wrow[t], wcols[t]])
                    for d in range(1, _ND):
                        p = p + accs[d] * plsc.load_gather(
                            w_rows, [wrow[t], wcols[t] + (d * _LG)])
                    if t > 0:
                        p = -p
                    out_v[e * _K1 + t, pl.ds(0, _LG)] = p
                return carry

            lax.fori_loop(0, _CH, elem, 0)
            pltpu.sync_copy(
                out_v, out_hbm.at[pl.ds((base + off) * _K1, _CH * _K1)])

    return body(u_half, u_offs, w_half, w_offs, u2, w2)


_TW = 20480
_SPLIT = 512000


def _tc_relayout_body(xa_ref, xb_ref, o_ref):
    d_ids = lax.broadcasted_iota(jnp.int32, (_D, _DP), 0)
    p_ids = lax.broadcasted_iota(jnp.int32, (_D, _DP), 1)
    e_lo = (d_ids == p_ids).astype(jnp.float32)
    e_hi = (d_ids == (p_ids - _D)).astype(jnp.float32)
    o_ref[...] = (
        jax.lax.dot_general(
            xa_ref[...], e_lo, (((0,), (0,)), ((), ())),
            preferred_element_type=jnp.float32)
        + jax.lax.dot_general(
            xb_ref[...], e_hi, (((0,), (0,)), ((), ())),
            preferred_element_type=jnp.float32))


def _tc_relayout(table):
    nblk = _SPLIT // _TW
    last = (table.shape[0] - 1) // _TW
    return pl.pallas_call(
        _tc_relayout_body,
        grid=(nblk,),
        in_specs=[
            pl.BlockSpec((_D, _TW), lambda j: (0, j)),
            pl.BlockSpec((_D, _TW),
                         lambda j: (0, jnp.minimum(j + nblk, last))),
        ],
        out_specs=pl.BlockSpec((_TW, _DP), lambda j: (j, 0)),
        out_shape=jax.ShapeDtypeStruct((_SPLIT, _DP), jnp.float32),
    )(table.T, table.T)


def _tc_loss_body(x_ref, o_ref):
    x = x_ref[...]
    col = lax.broadcasted_iota(jnp.int32, (128, 8), 0) // _LG
    grp = lax.broadcasted_iota(jnp.int32, (128, 8), 1)
    sel = (col == grp).astype(jnp.float32)
    s = jnp.dot(x, sel, preferred_element_type=jnp.float32)
    ls = jnp.minimum(s, 0.0) - jnp.log1p(jnp.exp(-jnp.abs(s)))
    o_ref[0, 0] = -jnp.sum(ls)


def _tc_loss(lanes_flat):
    return pl.pallas_call(
        _tc_loss_body,
        out_shape=jax.ShapeDtypeStruct((1, 1), jnp.float32),
        in_specs=[pl.BlockSpec(memory_space=pltpu.VMEM)],
        out_specs=pl.BlockSpec(memory_space=pltpu.SMEM),
    )(lanes_flat)


def kernel(pos_u, pos_w, neg_w, u_weight, w_weight):
    u_idx = pos_u.astype(jnp.int32).reshape(_B * _C)
    w_idx = jnp.concatenate(
        [pos_w.astype(jnp.int32)[:, None], neg_w.astype(jnp.int32)],
        axis=1).reshape(_B * _K1)
    u2 = _tc_relayout(u_weight)
    w2 = _tc_relayout(w_weight)
    u_half = jnp.where(u_idx >= _SPLIT, u_idx - _SPLIT, u_idx)
    w_half = jnp.where(w_idx >= _SPLIT, w_idx - _SPLIT, w_idx)
    lanes16 = jnp.arange(_LG, dtype=jnp.int32)[None, :]
    u_offs = (jnp.where(u_idx >= _SPLIT, _D, 0)[:, None]
              + lanes16).reshape(_B * _C * _LG)
    w_offs = (jnp.where(w_idx >= _SPLIT, _D, 0)[:, None]
              + lanes16).reshape(_B * _K1 * _LG)
    lanes = _sc_scores(u_half, u_offs, w_half, w_offs, u2, w2)
    return _tc_loss(lanes.reshape(_ROWS, 128))[0, 0]


# speedup vs baseline: 3.3628x; 1.0044x over previous
import functools

import jax
import jax.numpy as jnp
from jax import lax
from jax.experimental import pallas as pl
from jax.experimental.pallas import tpu as pltpu
from jax.experimental.pallas import tpu_sc as plsc

_B, _C, _K1, _D = 4096, 10, 6, 64
_NW = 32
_BPW = _B // _NW
_HALF = _BPW // 2
_LG = 16
_ND = _D // _LG
_ROWS = _B * _K1 * _LG // 128


_CH = 32
_NROUND = _BPW // _CH
_DP = 128


def _streams(total):
    out, off = [], 0
    while off < total:
        n = min(128, total - off)
        out.append((off, n))
        off += n
    return out


def _sc_scores(u_half, u_offs, w_half, w_offs, u2, w2):
    mesh = plsc.VectorSubcoreMesh(core_axis_name="c", subcore_axis_name="s")

    @functools.partial(
        pl.kernel,
        out_type=jax.ShapeDtypeStruct((_B * _K1, _LG), jnp.float32),
        mesh=mesh,
        scratch_types=[
            pltpu.VMEM((_BPW * _C,), jnp.int32),
            pltpu.VMEM((_BPW * _K1,), jnp.int32),
            pltpu.VMEM((_BPW * _C * _LG,), jnp.int32),
            pltpu.VMEM((_BPW * _K1 * _LG,), jnp.int32),
            pltpu.VMEM((_CH * _C, _DP), jnp.float32),
            pltpu.VMEM((_CH * _K1, _DP), jnp.float32),
            pltpu.VMEM((_CH * _K1, _LG), jnp.float32),
            pltpu.SemaphoreType.DMA,
        ],
        compiler_params=pltpu.CompilerParams(needs_layout_passes=False),
    )
    def body(uh_hbm, uo_hbm, wh_hbm, wo_hbm, ut_hbm, wt_hbm, out_hbm,
             uh_v, wh_v, uo_v, wo_v, u_rows, w_rows, out_v, sem):
        wid = lax.axis_index("s") * 2 + lax.axis_index("c")
        base = wid * _BPW
        pltpu.sync_copy(uh_hbm.at[pl.ds(base * _C, _BPW * _C)], uh_v)
        pltpu.sync_copy(wh_hbm.at[pl.ds(base * _K1, _BPW * _K1)], wh_v)
        pltpu.sync_copy(
            uo_hbm.at[pl.ds(base * _C * _LG, _BPW * _C * _LG)], uo_v)
        pltpu.sync_copy(
            wo_hbm.at[pl.ds(base * _K1 * _LG, _BPW * _K1 * _LG)], wo_v)

        for rnd in range(_NROUND):
            off = rnd * _CH
            copies = []
            for (so, sn) in _streams(_CH * _C):
                copies.append(pltpu.async_copy(
                    ut_hbm.at[uh_v.at[pl.ds(off * _C + so, sn)]],
                    u_rows.at[pl.ds(so, sn)], sem))
            for (so, sn) in _streams(_CH * _K1):
                copies.append(pltpu.async_copy(
                    wt_hbm.at[wh_v.at[pl.ds(off * _K1 + so, sn)]],
                    w_rows.at[pl.ds(so, sn)], sem))
            for cp in copies:
                cp.wait()

            zeros16 = jnp.zeros((_LG,), jnp.int32)

            def elem(e, carry):
                ucols = [uo_v[pl.ds(((off + e) * _C + c) * _LG, _LG)]
                         for c in range(_C)]
                wcols = [wo_v[pl.ds(((off + e) * _K1 + t) * _LG, _LG)]
                         for t in range(_K1)]
                urow = [zeros16 + (e * _C + c) for c in range(_C)]
                wrow = [zeros16 + (e * _K1 + t) for t in range(_K1)]
                accs = []
                for d in range(_ND):
                    a = plsc.load_gather(
                        u_rows, [urow[0], ucols[0] + (d * _LG)])
                    for c in range(1, _C):
                        a = a + plsc.load_gather(
                            u_rows, [urow[c], ucols[c] + (d * _LG)])
                    accs.append(a)
                for t in range(_K1):
                    p = accs[0] * plsc.load_gather(
                        w_rows, [wrow[t], wcols[t]])
                    for d in range(1, _ND):
                        p = p + accs[d] * plsc.load_gather(
                            w_rows, [wrow[t], wcols[t] + (d * _LG)])
                    if t > 0:
                        p = -p
                    out_v[e * _K1 + t, pl.ds(0, _LG)] = p
                return carry

            lax.fori_loop(0, _CH, elem, 0)
            pltpu.sync_copy(
                out_v, out_hbm.at[pl.ds((base + off) * _K1, _CH * _K1)])

    return body(u_half, u_offs, w_half, w_offs, u2, w2)


_TW = 25600
_SPLIT = 512000


def _tc_relayout_body(xa_ref, xb_ref, o_ref):
    d_ids = lax.broadcasted_iota(jnp.int32, (_D, _DP), 0)
    p_ids = lax.broadcasted_iota(jnp.int32, (_D, _DP), 1)
    e_lo = (d_ids == p_ids).astype(jnp.float32)
    e_hi = (d_ids == (p_ids - _D)).astype(jnp.float32)
    o_ref[...] = (
        jax.lax.dot_general(
            xa_ref[...], e_lo, (((0,), (0,)), ((), ())),
            preferred_element_type=jnp.float32)
        + jax.lax.dot_general(
            xb_ref[...], e_hi, (((0,), (0,)), ((), ())),
            preferred_element_type=jnp.float32))


def _tc_relayout(table):
    nblk = _SPLIT // _TW
    last = (table.shape[0] - 1) // _TW
    return pl.pallas_call(
        _tc_relayout_body,
        grid=(nblk,),
        in_specs=[
            pl.BlockSpec((_D, _TW), lambda j: (0, j)),
            pl.BlockSpec((_D, _TW),
                         lambda j: (0, jnp.minimum(j + nblk, last))),
        ],
        out_specs=pl.BlockSpec((_TW, _DP), lambda j: (j, 0)),
        out_shape=jax.ShapeDtypeStruct((_SPLIT, _DP), jnp.float32),
        compiler_params=pltpu.CompilerParams(
            vmem_limit_bytes=63 * 1024 * 1024),
    )(table.T, table.T)


def _tc_loss_body(x_ref, o_ref):
    x = x_ref[...]
    col = lax.broadcasted_iota(jnp.int32, (128, 8), 0) // _LG
    grp = lax.broadcasted_iota(jnp.int32, (128, 8), 1)
    sel = (col == grp).astype(jnp.float32)
    s = jnp.dot(x, sel, preferred_element_type=jnp.float32)
    ls = jnp.minimum(s, 0.0) - jnp.log1p(jnp.exp(-jnp.abs(s)))
    o_ref[0, 0] = -jnp.sum(ls)


def _tc_loss(lanes_flat):
    return pl.pallas_call(
        _tc_loss_body,
        out_shape=jax.ShapeDtypeStruct((1, 1), jnp.float32),
        in_specs=[pl.BlockSpec(memory_space=pltpu.VMEM)],
        out_specs=pl.BlockSpec(memory_space=pltpu.SMEM),
    )(lanes_flat)


def kernel(pos_u, pos_w, neg_w, u_weight, w_weight):
    u_idx = pos_u.astype(jnp.int32).reshape(_B * _C)
    w_idx = jnp.concatenate(
        [pos_w.astype(jnp.int32)[:, None], neg_w.astype(jnp.int32)],
        axis=1).reshape(_B * _K1)
    u2 = _tc_relayout(u_weight)
    w2 = _tc_relayout(w_weight)
    u_half = jnp.where(u_idx >= _SPLIT, u_idx - _SPLIT, u_idx)
    w_half = jnp.where(w_idx >= _SPLIT, w_idx - _SPLIT, w_idx)
    lanes16 = jnp.arange(_LG, dtype=jnp.int32)[None, :]
    u_offs = (jnp.where(u_idx >= _SPLIT, _D, 0)[:, None]
              + lanes16).reshape(_B * _C * _LG)
    w_offs = (jnp.where(w_idx >= _SPLIT, _D, 0)[:, None]
              + lanes16).reshape(_B * _K1 * _LG)
    lanes = _sc_scores(u_half, u_offs, w_half, w_offs, u2, w2)
    return _tc_loss(lanes.reshape(_ROWS, 128))[0, 0]
